# baseline (speedup 1.0000x reference)
import functools

import jax
import jax.numpy as jnp
from jax import lax
from jax.experimental import pallas as pl
from jax.experimental.pallas import tpu as pltpu
from jax.experimental.pallas import tpu_sc as plsc

VOCAB1 = 100001
EMBED_DIM = 8
BATCH = 16384


@functools.lru_cache(maxsize=None)
def _make_sc_gather(V, D, B):
    info = plsc.get_sparse_core_info()
    NC, NS, L = info.num_cores, info.num_subcores, info.num_lanes
    NW = NC * NS
    b_per_w = B // NW
    mesh = plsc.VectorSubcoreMesh(core_axis_name="c", subcore_axis_name="s")

    @functools.partial(
        pl.kernel,
        mesh=mesh,
        compiler_params=pltpu.CompilerParams(use_tc_tiling_on_sc=False),
        out_type=jax.ShapeDtypeStruct((D, B), jnp.float32),
        scratch_types=[
            pltpu.VMEM((b_per_w,), jnp.int32),
            pltpu.VMEM((D * b_per_w,), jnp.int32),
            pltpu.VMEM((D * b_per_w,), jnp.float32),
            pltpu.SemaphoreType.DMA,
        ],
    )
    def gather(tflat_hbm, idx_hbm, out_hbm, idx_v, fi_v, rows_v, sem):
        wid = lax.axis_index("s") * NC + lax.axis_index("c")
        base = wid * b_per_w
        pltpu.sync_copy(idx_hbm.at[pl.ds(base, b_per_w)], idx_v)
        for d in range(D):
            off = jnp.full((L,), d * V, jnp.int32)
            for j in range(b_per_w // L):
                fi_v[pl.ds(d * b_per_w + j * L, L)] = idx_v[pl.ds(j * L, L)] + off
        pltpu.async_copy(tflat_hbm.at[fi_v], rows_v, sem).wait()
        for d in range(D):
            pltpu.sync_copy(
                rows_v.at[pl.ds(d * b_per_w, b_per_w)],
                out_hbm.at[d, pl.ds(base, b_per_w)],
            )

    return gather


def _mlp_t_body(embt_ref, w1_ref, b1_ref, w2_ref, b2_ref, out_ref):
    embt = embt_ref[...].astype(jnp.bfloat16)
    w1 = w1_ref[...].astype(jnp.bfloat16)
    ht = lax.dot_general(w1, embt, (((0,), (0,)), ((), ())),
                         preferred_element_type=jnp.float32)
    ht = jnp.maximum(ht + b1_ref[...], 0.0)
    w2 = w2_ref[...].astype(jnp.bfloat16)
    outt = lax.dot_general(w2, ht.astype(jnp.bfloat16), (((0,), (0,)), ((), ())),
                           preferred_element_type=jnp.float32)
    out_ref[...] = outt + b2_ref[...]


def _mlp_t(embt, W1, b1, W2, b2, tile=4096):
    B = embt.shape[1]
    H = W1.shape[1]
    O = W2.shape[1]
    return pl.pallas_call(
        _mlp_t_body,
        grid=(B // tile,),
        in_specs=[
            pl.BlockSpec((EMBED_DIM, tile), lambda i: (0, i)),
            pl.BlockSpec((EMBED_DIM, H), lambda i: (0, 0)),
            pl.BlockSpec((H, 1), lambda i: (0, 0)),
            pl.BlockSpec((H, O), lambda i: (0, 0)),
            pl.BlockSpec((O, 1), lambda i: (0, 0)),
        ],
        out_specs=pl.BlockSpec((O, tile), lambda i: (0, i)),
        out_shape=jax.ShapeDtypeStruct((O, B), jnp.float32),
    )(embt, W1, b1.reshape(H, 1), W2, b2.reshape(O, 1))


def kernel(indices, table, W1, b1, W2, b2):
    tflat = table.T.reshape(-1)
    embt = _make_sc_gather(VOCAB1, EMBED_DIM, BATCH)(tflat, indices)
    outt = _mlp_t(embt, W1, b1, W2, b2)
    return outt.T


# speedup vs baseline: 1.0043x; 1.0043x over previous
import functools

import jax
import jax.numpy as jnp
from jax import lax
from jax.experimental import pallas as pl
from jax.experimental.pallas import tpu as pltpu
from jax.experimental.pallas import tpu_sc as plsc

VOCAB1 = 100001
EMBED_DIM = 8
BATCH = 16384


@functools.lru_cache(maxsize=None)
def _make_sc_gather(V, D, B):
    info = plsc.get_sparse_core_info()
    NC, NS, L = info.num_cores, info.num_subcores, info.num_lanes
    NW = NC * NS
    b_per_w = B // NW
    mesh = plsc.VectorSubcoreMesh(core_axis_name="c", subcore_axis_name="s")

    @functools.partial(
        pl.kernel,
        mesh=mesh,
        compiler_params=pltpu.CompilerParams(use_tc_tiling_on_sc=False),
        out_type=jax.ShapeDtypeStruct((D, B), jnp.float32),
        scratch_types=[
            pltpu.VMEM((b_per_w,), jnp.int32),
            pltpu.VMEM((D * b_per_w,), jnp.int32),
            pltpu.VMEM((D * b_per_w,), jnp.float32),
            pltpu.SemaphoreType.DMA,
        ],
    )
    def gather(tflat_hbm, idx_hbm, out_hbm, idx_v, fi_v, rows_v, sem):
        wid = lax.axis_index("s") * NC + lax.axis_index("c")
        base = wid * b_per_w
        pltpu.sync_copy(idx_hbm.at[pl.ds(base, b_per_w)], idx_v)
        for d in range(D):
            off = jnp.full((L,), d * V, jnp.int32)
            for j in range(b_per_w // L):
                fi_v[pl.ds(d * b_per_w + j * L, L)] = idx_v[pl.ds(j * L, L)] + off
        pltpu.async_copy(tflat_hbm.at[fi_v], rows_v, sem).wait()
        for d in range(D):
            pltpu.sync_copy(
                rows_v.at[pl.ds(d * b_per_w, b_per_w)],
                out_hbm.at[d, pl.ds(base, b_per_w)],
            )

    return gather


def _mlp_t_body(embt_ref, w1_ref, b1_ref, w2t_ref, b2_ref, out_ref):
    embt = embt_ref[...].astype(jnp.bfloat16)
    w1 = w1_ref[...].astype(jnp.bfloat16)
    ht = lax.dot_general(w1, embt, (((0,), (0,)), ((), ())),
                         preferred_element_type=jnp.float32)
    ht = jnp.maximum(ht + b1_ref[...], 0.0)
    w2t = w2t_ref[...].astype(jnp.bfloat16)
    outt = lax.dot_general(w2t, ht.astype(jnp.bfloat16), (((1,), (0,)), ((), ())),
                           preferred_element_type=jnp.float32)
    out_ref[...] = outt + b2_ref[...]


def _mlp_t(embt, W1, b1, W2, b2, tile=8192):
    B = embt.shape[1]
    H = W1.shape[1]
    O = W2.shape[1]
    return pl.pallas_call(
        _mlp_t_body,
        grid=(B // tile,),
        in_specs=[
            pl.BlockSpec((EMBED_DIM, tile), lambda i: (0, i)),
            pl.BlockSpec((EMBED_DIM, H), lambda i: (0, 0)),
            pl.BlockSpec((H, 1), lambda i: (0, 0)),
            pl.BlockSpec((O, H), lambda i: (0, 0)),
            pl.BlockSpec((O, 1), lambda i: (0, 0)),
        ],
        out_specs=pl.BlockSpec((O, tile), lambda i: (0, i)),
        out_shape=jax.ShapeDtypeStruct((O, B), jnp.float32),
    )(embt, W1, b1.reshape(H, 1), W2.T, b2.reshape(O, 1))


def kernel(indices, table, W1, b1, W2, b2):
    tflat = table.T.reshape(-1)
    embt = _make_sc_gather(VOCAB1, EMBED_DIM, BATCH)(tflat, indices)
    outt = _mlp_t(embt, W1, b1, W2, b2)
    return outt.T


# speedup vs baseline: 1.0762x; 1.0716x over previous
import functools

import jax
import jax.numpy as jnp
from jax import lax
from jax.experimental import pallas as pl
from jax.experimental.pallas import tpu as pltpu
from jax.experimental.pallas import tpu_sc as plsc

VOCAB1 = 100001
EMBED_DIM = 8
BATCH = 16384


@functools.lru_cache(maxsize=None)
def _make_sc_gather(V, D, B):
    info = plsc.get_sparse_core_info()
    NC, NS, L = info.num_cores, info.num_subcores, info.num_lanes
    NW = NC * NS
    b_per_w = B // NW
    mesh = plsc.VectorSubcoreMesh(core_axis_name="c", subcore_axis_name="s")

    @functools.partial(
        pl.kernel,
        mesh=mesh,
        compiler_params=pltpu.CompilerParams(use_tc_tiling_on_sc=False),
        out_type=jax.ShapeDtypeStruct((B * D,), jnp.float32),
        scratch_types=[
            pltpu.VMEM((b_per_w,), jnp.int32),
            pltpu.VMEM((D * b_per_w,), jnp.int32),
            pltpu.VMEM((D * b_per_w,), jnp.float32),
            pltpu.SemaphoreType.DMA,
        ],
    )
    def gather(tflat_hbm, idx_hbm, out_hbm, idx_v, fi_v, rows_v, sem):
        wid = lax.axis_index("s") * NC + lax.axis_index("c")
        base = wid * b_per_w
        t_per_w = b_per_w // 128
        pltpu.sync_copy(idx_hbm.at[pl.ds(base, b_per_w)], idx_v)
        for k in range(t_per_w):
            for d in range(D):
                off = jnp.full((L,), d * V, jnp.int32)
                for j in range(128 // L):
                    fi_v[pl.ds((k * D + d) * 128 + j * L, L)] = (
                        idx_v[pl.ds(k * 128 + j * L, L)] + off
                    )
        pltpu.async_copy(tflat_hbm.at[fi_v], rows_v, sem).wait()
        pltpu.sync_copy(rows_v, out_hbm.at[pl.ds(base * D, b_per_w * D)])

    return gather


def _mlp_t_body(embt_ref, w1_ref, b1_ref, w2t_ref, b2_ref, out_ref):
    embt = embt_ref[...].astype(jnp.bfloat16)
    w1 = w1_ref[...].astype(jnp.bfloat16)
    ht = lax.dot_general(w1, embt, (((0,), (0,)), ((), ())),
                         preferred_element_type=jnp.float32)
    ht = jnp.maximum(ht + b1_ref[...], 0.0)
    w2t = w2t_ref[...].astype(jnp.bfloat16)
    outt = lax.dot_general(w2t, ht.astype(jnp.bfloat16), (((1,), (0,)), ((), ())),
                           preferred_element_type=jnp.float32)
    out_ref[...] = outt + b2_ref[...]


def _mlp_t(embt, W1, b1, W2, b2, tile=8192):
    B = embt.shape[1]
    H = W1.shape[1]
    O = W2.shape[1]
    return pl.pallas_call(
        _mlp_t_body,
        grid=(B // tile,),
        in_specs=[
            pl.BlockSpec((EMBED_DIM, tile), lambda i: (0, i)),
            pl.BlockSpec((EMBED_DIM, H), lambda i: (0, 0)),
            pl.BlockSpec((H, 1), lambda i: (0, 0)),
            pl.BlockSpec((O, H), lambda i: (0, 0)),
            pl.BlockSpec((O, 1), lambda i: (0, 0)),
        ],
        out_specs=pl.BlockSpec((O, tile), lambda i: (0, i)),
        out_shape=jax.ShapeDtypeStruct((O, B), jnp.float32),
    )(embt, W1, b1.reshape(H, 1), W2.T, b2.reshape(O, 1))


def kernel(indices, table, W1, b1, W2, b2):
    tflat = table.T.reshape(-1)
    flat = _make_sc_gather(VOCAB1, EMBED_DIM, BATCH)(tflat, indices)
    embt3 = flat.reshape(BATCH // 128, EMBED_DIM, 128)
    embt = embt3.transpose(1, 0, 2).reshape(EMBED_DIM, BATCH)
    outt = _mlp_t(embt, W1, b1, W2, b2)
    return outt.T


# speedup vs baseline: 1.1024x; 1.0244x over previous
import functools

import jax
import jax.numpy as jnp
from jax import lax
from jax.experimental import pallas as pl
from jax.experimental.pallas import tpu as pltpu
from jax.experimental.pallas import tpu_sc as plsc

VOCAB1 = 100001
EMBED_DIM = 8
BATCH = 16384


@functools.lru_cache(maxsize=None)
def _make_sc_gather(V, D, B):
    info = plsc.get_sparse_core_info()
    NC, NS, L = info.num_cores, info.num_subcores, info.num_lanes
    NW = NC * NS
    b_per_w = B // NW
    mesh = plsc.VectorSubcoreMesh(core_axis_name="c", subcore_axis_name="s")

    @functools.partial(
        pl.kernel,
        mesh=mesh,
        compiler_params=pltpu.CompilerParams(use_tc_tiling_on_sc=False),
        out_type=jax.ShapeDtypeStruct((B * D,), jnp.float32),
        scratch_types=[
            pltpu.VMEM((b_per_w,), jnp.int32),
            pltpu.VMEM((D * b_per_w,), jnp.int32),
            pltpu.VMEM((D * b_per_w,), jnp.float32),
            pltpu.SemaphoreType.DMA,
        ],
    )
    def gather(tflat_hbm, idx_hbm, out_hbm, idx_v, fi_v, rows_v, sem):
        wid = lax.axis_index("s") * NC + lax.axis_index("c")
        base = wid * b_per_w
        t_per_w = b_per_w // 128
        pltpu.sync_copy(idx_hbm.at[pl.ds(base, b_per_w)], idx_v)
        for k in range(t_per_w):
            for j in range(128 // L):
                iv = idx_v[pl.ds(k * 128 + j * L, L)]
                tbase = lax.shift_right_logical(iv, 7) * 1024 + lax.rem(iv, 128)
                for d in range(D):
                    fi_v[pl.ds((k * D + d) * 128 + j * L, L)] = tbase + d * 128
        pltpu.async_copy(tflat_hbm.at[fi_v], rows_v, sem).wait()
        pltpu.sync_copy(rows_v, out_hbm.at[pl.ds(base * D, b_per_w * D)])

    return gather


def _mlp_t_body(embt_ref, w1_ref, b1_ref, w2t_ref, b2_ref, out_ref):
    embt = embt_ref[...].astype(jnp.bfloat16)
    w1 = w1_ref[...].astype(jnp.bfloat16)
    ht = lax.dot_general(w1, embt, (((0,), (0,)), ((), ())),
                         preferred_element_type=jnp.float32)
    ht = jnp.maximum(ht + b1_ref[...], 0.0)
    w2t = w2t_ref[...].astype(jnp.bfloat16)
    outt = lax.dot_general(w2t, ht.astype(jnp.bfloat16), (((1,), (0,)), ((), ())),
                           preferred_element_type=jnp.float32)
    out_ref[...] = outt + b2_ref[...]


def _mlp_t(embt, W1, b1, W2, b2, tile=16384):
    B = embt.shape[1]
    H = W1.shape[1]
    O = W2.shape[1]
    return pl.pallas_call(
        _mlp_t_body,
        grid=(B // tile,),
        in_specs=[
            pl.BlockSpec((EMBED_DIM, tile), lambda i: (0, i)),
            pl.BlockSpec((EMBED_DIM, H), lambda i: (0, 0)),
            pl.BlockSpec((H, 1), lambda i: (0, 0)),
            pl.BlockSpec((O, H), lambda i: (0, 0)),
            pl.BlockSpec((O, 1), lambda i: (0, 0)),
        ],
        out_specs=pl.BlockSpec((O, tile), lambda i: (0, i)),
        out_shape=jax.ShapeDtypeStruct((O, B), jnp.float32),
    )(embt, W1, b1.reshape(H, 1), W2.T, b2.reshape(O, 1))


def kernel(indices, table, W1, b1, W2, b2):
    vpad = -VOCAB1 % 128
    tflat = (
        jnp.pad(table, ((0, vpad), (0, 0)))
        .T.reshape(EMBED_DIM, (VOCAB1 + vpad) // 128, 128)
        .transpose(1, 0, 2)
        .reshape(-1)
    )
    flat = _make_sc_gather(VOCAB1, EMBED_DIM, BATCH)(tflat, indices)
    embt3 = flat.reshape(BATCH // 128, EMBED_DIM, 128)
    embt = embt3.transpose(1, 0, 2).reshape(EMBED_DIM, BATCH)
    outt = _mlp_t(embt, W1, b1, W2, b2)
    return outt.T


# speedup vs baseline: 1.1272x; 1.0225x over previous
import functools

import jax
import jax.numpy as jnp
from jax import lax
from jax.experimental import pallas as pl
from jax.experimental.pallas import tpu as pltpu
from jax.experimental.pallas import tpu_sc as plsc

VOCAB1 = 100001
EMBED_DIM = 8
BATCH = 16384


@functools.lru_cache(maxsize=None)
def _make_sc_gather(V, D, B):
    info = plsc.get_sparse_core_info()
    NC, NS, L = info.num_cores, info.num_subcores, info.num_lanes
    NW = NC * NS
    b_per_w = B // NW
    mesh = plsc.VectorSubcoreMesh(core_axis_name="c", subcore_axis_name="s")

    @functools.partial(
        pl.kernel,
        mesh=mesh,
        compiler_params=pltpu.CompilerParams(use_tc_tiling_on_sc=False),
        out_type=jax.ShapeDtypeStruct((B * D,), jnp.float32),
        scratch_types=[
            pltpu.VMEM((b_per_w,), jnp.int32),
            pltpu.VMEM((D * b_per_w,), jnp.int32),
            pltpu.VMEM((D * b_per_w,), jnp.float32),
            pltpu.SemaphoreType.DMA,
        ],
    )
    def gather(tflat_hbm, idx_hbm, out_hbm, idx_v, fi_v, rows_v, sem):
        wid = lax.axis_index("s") * NC + lax.axis_index("c")
        base = wid * b_per_w
        t_per_w = b_per_w // 128
        pltpu.sync_copy(idx_hbm.at[pl.ds(base, b_per_w)], idx_v)
        def chunk(c, _):
            iv = idx_v[pl.ds(c * L, L)]
            tbase = (
                lax.shift_left(lax.shift_right_logical(iv, 7), 10)
                + lax.bitwise_and(iv, 127)
            )
            k = c // (128 // L)
            j = c % (128 // L)
            for d in range(D):
                fi_v[pl.ds(k * (D * 128) + d * 128 + j * L, L)] = tbase + d * 128
            return _

        lax.fori_loop(0, b_per_w // L, chunk, 0)
        pltpu.async_copy(tflat_hbm.at[fi_v], rows_v, sem).wait()
        pltpu.sync_copy(rows_v, out_hbm.at[pl.ds(base * D, b_per_w * D)])

    return gather


def _mlp_t_body(embt_ref, w1_ref, b1_ref, w2t_ref, b2_ref, out_ref):
    embt = embt_ref[...].astype(jnp.bfloat16)
    w1 = w1_ref[...].astype(jnp.bfloat16)
    ht = lax.dot_general(w1, embt, (((0,), (0,)), ((), ())),
                         preferred_element_type=jnp.float32)
    ht = jnp.maximum(ht + b1_ref[...], 0.0)
    w2t = w2t_ref[...].astype(jnp.bfloat16)
    outt = lax.dot_general(w2t, ht.astype(jnp.bfloat16), (((1,), (0,)), ((), ())),
                           preferred_element_type=jnp.float32)
    out_ref[...] = outt + b2_ref[...]


def _mlp_t(embt, W1, b1, W2, b2, tile=8192):
    B = embt.shape[1]
    H = W1.shape[1]
    O = W2.shape[1]
    return pl.pallas_call(
        _mlp_t_body,
        grid=(B // tile,),
        in_specs=[
            pl.BlockSpec((EMBED_DIM, tile), lambda i: (0, i)),
            pl.BlockSpec((EMBED_DIM, H), lambda i: (0, 0)),
            pl.BlockSpec((H, 1), lambda i: (0, 0)),
            pl.BlockSpec((O, H), lambda i: (0, 0)),
            pl.BlockSpec((O, 1), lambda i: (0, 0)),
        ],
        out_specs=pl.BlockSpec((O, tile), lambda i: (0, i)),
        out_shape=jax.ShapeDtypeStruct((O, B), jnp.float32),
    )(embt, W1, b1.reshape(H, 1), W2.T, b2.reshape(O, 1))


def kernel(indices, table, W1, b1, W2, b2):
    vpad = -VOCAB1 % 128
    tflat = (
        jnp.pad(table, ((0, vpad), (0, 0)))
        .T.reshape(EMBED_DIM, (VOCAB1 + vpad) // 128, 128)
        .transpose(1, 0, 2)
        .reshape(-1)
    )
    flat = _make_sc_gather(VOCAB1, EMBED_DIM, BATCH)(tflat, indices)
    embt3 = flat.reshape(BATCH // 128, EMBED_DIM, 128)
    embt = embt3.transpose(1, 0, 2).reshape(EMBED_DIM, BATCH)
    outt = _mlp_t(embt, W1, b1, W2, b2)
    return outt.T


# speedup vs baseline: 1.1275x; 1.0002x over previous
import functools

import jax
import jax.numpy as jnp
from jax import lax
from jax.experimental import pallas as pl
from jax.experimental.pallas import tpu as pltpu
from jax.experimental.pallas import tpu_sc as plsc

VOCAB1 = 100001
EMBED_DIM = 8
BATCH = 16384


@functools.lru_cache(maxsize=None)
def _make_sc_gather(V, D, B):
    info = plsc.get_sparse_core_info()
    NC, NS, L = info.num_cores, info.num_subcores, info.num_lanes
    NW = NC * NS
    b_per_w = B // NW
    mesh = plsc.VectorSubcoreMesh(core_axis_name="c", subcore_axis_name="s")

    @functools.partial(
        pl.kernel,
        mesh=mesh,
        compiler_params=pltpu.CompilerParams(use_tc_tiling_on_sc=False),
        out_type=jax.ShapeDtypeStruct((B * D,), jnp.float32),
        scratch_types=[
            pltpu.VMEM((b_per_w,), jnp.int32),
            pltpu.VMEM((D * b_per_w,), jnp.int32),
            pltpu.VMEM((D * b_per_w,), jnp.float32),
            pltpu.SemaphoreType.DMA,
        ],
    )
    def gather(tflat_hbm, idx_hbm, out_hbm, idx_v, fi_v, rows_v, sem):
        wid = lax.axis_index("s") * NC + lax.axis_index("c")
        base = wid * b_per_w
        pltpu.sync_copy(idx_hbm.at[pl.ds(base, b_per_w)], idx_v)
        def chunk(c, _):
            iv = idx_v[pl.ds(c * L, L)]
            tbase = (
                lax.shift_left(lax.shift_right_logical(iv, 7), 10)
                + lax.bitwise_and(iv, 127)
            )
            k = c // (128 // L)
            j = c % (128 // L)
            for d in range(D):
                fi_v[pl.ds(k * (D * 128) + d * 128 + j * L, L)] = tbase + d * 128
            return _

        lax.fori_loop(0, b_per_w // L, chunk, 0)
        pltpu.async_copy(tflat_hbm.at[fi_v], rows_v, sem).wait()
        pltpu.sync_copy(rows_v, out_hbm.at[pl.ds(base * D, b_per_w * D)])

    return gather


def _mlp_t_body(embt_ref, w1_ref, b1_ref, w2t_ref, b2_ref, out_ref):
    embt = embt_ref[...].astype(jnp.bfloat16)
    w1 = w1_ref[...].astype(jnp.bfloat16)
    ht = lax.dot_general(w1, embt, (((0,), (0,)), ((), ())),
                         preferred_element_type=jnp.float32)
    ht = jnp.maximum(ht + b1_ref[...], 0.0)
    w2t = w2t_ref[...].astype(jnp.bfloat16)
    outt = lax.dot_general(w2t, ht.astype(jnp.bfloat16), (((1,), (0,)), ((), ())),
                           preferred_element_type=jnp.float32)
    out_ref[...] = outt + b2_ref[...]


def _mlp_t(embt, W1, b1, W2, b2, tile=8192):
    B = embt.shape[1]
    H = W1.shape[1]
    O = W2.shape[1]
    return pl.pallas_call(
        _mlp_t_body,
        grid=(B // tile,),
        in_specs=[
            pl.BlockSpec((EMBED_DIM, tile), lambda i: (0, i)),
            pl.BlockSpec((EMBED_DIM, H), lambda i: (0, 0)),
            pl.BlockSpec((H, 1), lambda i: (0, 0)),
            pl.BlockSpec((O, H), lambda i: (0, 0)),
            pl.BlockSpec((O, 1), lambda i: (0, 0)),
        ],
        out_specs=pl.BlockSpec((O, tile), lambda i: (0, i)),
        out_shape=jax.ShapeDtypeStruct((O, B), jnp.float32),
    )(embt, W1, b1.reshape(H, 1), W2.T, b2.reshape(O, 1))


def kernel(indices, table, W1, b1, W2, b2):
    vpad = -VOCAB1 % 128
    tflat = (
        jnp.pad(table, ((0, vpad), (0, 0)))
        .T.reshape(EMBED_DIM, (VOCAB1 + vpad) // 128, 128)
        .transpose(1, 0, 2)
        .reshape(-1)
    )
    flat = _make_sc_gather(VOCAB1, EMBED_DIM, BATCH)(tflat, indices)
    embt3 = flat.reshape(BATCH // 128, EMBED_DIM, 128)
    embt = embt3.transpose(1, 0, 2).reshape(EMBED_DIM, BATCH)
    outt = _mlp_t(embt, W1, b1, W2, b2)
    return outt.T
